# baseline (speedup 1.0000x reference)
import functools
import math

import jax
import jax.numpy as jnp
from jax import lax
from jax.experimental import pallas as pl
from jax.experimental.pallas import tpu as pltpu
from jax.experimental.pallas import tpu_sc as plsc

NC = 2
NS = 16
NW = NC * NS
L = 16
DIM = 32
SCALE = math.sqrt(float(DIM))
CHUNK = 1600


@functools.partial(jax.jit, static_argnames=("n_total",))
def _sc_embed(xf, table, n_total):
    b_per_w = n_total // NW
    n_chunks = b_per_w // CHUNK
    mesh = plsc.VectorSubcoreMesh(
        core_axis_name="c", subcore_axis_name="s",
        num_cores=NC, num_subcores=NS)

    @functools.partial(
        pl.kernel,
        mesh=mesh,
        out_type=jax.ShapeDtypeStruct((n_total, DIM), jnp.float32),
        scratch_types=[
            pltpu.VMEM((CHUNK,), jnp.int32),
            pltpu.VMEM((CHUNK, DIM), jnp.float32),
            pltpu.SemaphoreType.DMA,
        ],
        compiler_params=pltpu.CompilerParams(use_tc_tiling_on_sc=False),
    )
    def k(x_hbm, table_hbm, out_hbm, idx_v, rows_v, sem):
        wid = lax.axis_index("s") * NC + lax.axis_index("c")
        base = wid * b_per_w

        def chunk_body(c, carry):
            off = pl.multiple_of(base + c * CHUNK, 8)
            pltpu.sync_copy(x_hbm.at[pl.ds(off, CHUNK)], idx_v)
            pltpu.async_copy(table_hbm.at[idx_v], rows_v, sem).wait()

            def row_body(r, carry2):
                for h in range(DIM // L):
                    sl = pl.ds(h * L, L)
                    rows_v[r, sl] = rows_v[r, sl] * SCALE
                return carry2

            lax.fori_loop(0, CHUNK, row_body, 0)
            pltpu.sync_copy(rows_v, out_hbm.at[pl.ds(off, CHUNK)])
            return carry

        lax.fori_loop(0, n_chunks, chunk_body, 0)

    return k(xf, table)


def kernel(x, table):
    xf = x.reshape(-1).astype(jnp.int32)
    out = _sc_embed(xf, table, xf.shape[0])
    return out.reshape(x.shape + (DIM,))


# speedup vs baseline: 1.5549x; 1.5549x over previous
import functools
import math

import jax
import jax.numpy as jnp
from jax import lax
from jax.experimental import pallas as pl
from jax.experimental.pallas import tpu as pltpu
from jax.experimental.pallas import tpu_sc as plsc

NC = 2
NS = 16
NW = NC * NS
L = 16
DIM = 32
SCALE = math.sqrt(float(DIM))
CHUNK = 1600


@functools.partial(jax.jit, static_argnames=("n_total",))
def _sc_embed(x, table, n_total):
    b_per_w = n_total // NW
    n_chunks = b_per_w // CHUNK
    mesh = plsc.VectorSubcoreMesh(
        core_axis_name="c", subcore_axis_name="s",
        num_cores=NC, num_subcores=NS)

    @functools.partial(
        pl.kernel,
        mesh=mesh,
        out_type=jax.ShapeDtypeStruct((n_total // 50, 50, DIM), jnp.float32),
        scratch_types=[
            pltpu.VMEM((CHUNK,), jnp.int32),
            pltpu.VMEM((CHUNK, DIM), jnp.float32),
            pltpu.SemaphoreType.DMA,
            pltpu.SemaphoreType.DMA,
        ],
        compiler_params=pltpu.CompilerParams(use_tc_tiling_on_sc=False),
    )
    def k(x_hbm, table_hbm, out_hbm, idx_v, rows_v, sem, osem):
        xrows = CHUNK // 50
        wid = lax.axis_index("s") * NC + lax.axis_index("c")

        def chunk_body(c, carry):
            off = pl.multiple_of((wid * n_chunks + c) * CHUNK, 8)
            pltpu.sync_copy(x_hbm.at[pl.ds(off, CHUNK)], idx_v)
            pltpu.async_copy(table_hbm.at[idx_v], rows_v, sem).wait()

            def row_body(r, carry2):
                for h in range(DIM // L):
                    sl = pl.ds(h * L, L)
                    rows_v[r, sl] = rows_v[r, sl] * SCALE
                return carry2

            lax.fori_loop(0, CHUNK, row_body, 0)
            r0 = (wid * n_chunks + c) * xrows
            copies = [
                pltpu.async_copy(rows_v.at[pl.ds(j * 50, 50)],
                                 out_hbm.at[r0 + j], osem)
                for j in range(xrows)
            ]
            for cpy in copies:
                cpy.wait()
            return carry

        lax.fori_loop(0, n_chunks, chunk_body, 0)

    return k(x, table)


def kernel(x, table):
    xf = x.reshape(-1)
    return _sc_embed(xf, table, xf.shape[0]).reshape(x.shape + (DIM,))


# speedup vs baseline: 1.7782x; 1.1436x over previous
import functools
import math

import jax
import jax.numpy as jnp
from jax import lax
from jax.experimental import pallas as pl
from jax.experimental.pallas import tpu as pltpu
from jax.experimental.pallas import tpu_sc as plsc

NC = 2
NS = 16
NW = NC * NS
L = 16
DIM = 32
SCALE = math.sqrt(float(DIM))
CHUNK = 800
XROWS = CHUNK // 50


@functools.partial(jax.jit, static_argnames=("n_total",))
def _sc_embed(x, table, n_total):
    b_per_w = n_total // NW
    n_chunks = b_per_w // CHUNK
    mesh = plsc.VectorSubcoreMesh(
        core_axis_name="c", subcore_axis_name="s",
        num_cores=NC, num_subcores=NS)

    @functools.partial(
        pl.kernel,
        mesh=mesh,
        out_type=jax.ShapeDtypeStruct((50, DIM, n_total // 50), jnp.float32),
        scratch_types=[
            pltpu.VMEM((b_per_w,), jnp.int32),
            pltpu.VMEM((CHUNK, DIM), jnp.float32),
            pltpu.VMEM((CHUNK, DIM), jnp.float32),
            pltpu.VMEM((50, DIM, XROWS), jnp.float32),
            pltpu.VMEM((50, DIM, XROWS), jnp.float32),
            pltpu.SemaphoreType.DMA,
            pltpu.SemaphoreType.DMA,
            pltpu.SemaphoreType.DMA,
            pltpu.SemaphoreType.DMA,
        ],
        compiler_params=pltpu.CompilerParams(use_tc_tiling_on_sc=False,
                                             needs_layout_passes=False),
    )
    def k(x_hbm, table_hbm, out_hbm, idx_v, rows0, rows1, tv0, tv1,
          g0, g1, o0, o1):
        rows = (rows0, rows1)
        tvs = (tv0, tv1)
        gsem = (g0, g1)
        osem = (o0, o1)
        wid = lax.axis_index("s") * NC + lax.axis_index("c")
        base = wid * b_per_w
        lanes = lax.iota(jnp.int32, L)

        pltpu.sync_copy(x_hbm.at[pl.ds(pl.multiple_of(base, 8), b_per_w)],
                        idx_v)

        def idx_slice(c):
            return idx_v.at[pl.ds(c * CHUNK, CHUNK)]

        def start_gather(c, b):
            pltpu.async_copy(table_hbm.at[idx_slice(c)], rows[b], gsem[b])

        def wait_gather(b):
            pltpu.make_async_copy(table_hbm.at[idx_slice(0)], rows[b],
                                  gsem[b]).wait()

        def out_slice(c):
            r0 = wid * n_chunks * XROWS + c * XROWS
            return out_hbm.at[:, :, pl.ds(r0, XROWS)]

        def start_out(c, b):
            pltpu.async_copy(tvs[b], out_slice(c), osem[b])

        def wait_out(c, b):
            pltpu.make_async_copy(tvs[b], out_slice(c), osem[b]).wait()

        def transpose_scale(b):
            rv, tv = rows[b], tvs[b]

            def t_body(t, carry2):
                ridx = lanes * 50 + t

                @plsc.parallel_loop(0, DIM, unroll=8)
                def d_body(d):
                    v = plsc.load_gather(rv, [ridx, jnp.full((L,), d,
                                                             jnp.int32)])
                    tv[t, d, :] = v * SCALE

                return carry2

            lax.fori_loop(0, 50, t_body, 0)

        start_gather(0, 0)

        def outer(o, carry):
            for b in range(2):
                c = 2 * o + b
                wait_gather(b)

                @pl.when(c + 1 < n_chunks)
                def _():
                    start_gather(c + 1, 1 - b)

                @pl.when(c >= 2)
                def _():
                    wait_out(c - 2, b)

                transpose_scale(b)
                start_out(c, b)
            return carry

        lax.fori_loop(0, n_chunks // 2, outer, 0)
        wait_out(n_chunks - 2, 0)
        wait_out(n_chunks - 1, 1)

    return k(x, table)


_TW = 1920


def _lin_body(in_ref, out_ref):
    pack = 128 // DIM
    y = in_ref[...].T
    z = y.reshape(_TW // pack, pack, DIM)
    out_ref[...] = jnp.concatenate([z[:, f, :] for f in range(pack)], axis=-1)


def _tc_linearize(tT):
    v = tT.shape[1]
    return pl.pallas_call(
        _lin_body,
        grid=((v + _TW - 1) // _TW,),
        in_specs=[pl.BlockSpec((DIM, _TW), lambda i: (0, i))],
        out_specs=pl.BlockSpec((_TW * DIM // 128, 128), lambda i: (i, 0)),
        out_shape=jax.ShapeDtypeStruct((v * DIM // 128, 128), jnp.float32),
    )(tT)


def kernel(x, table):
    xf = x.reshape(-1)
    t_lin = _tc_linearize(table.T).reshape(table.shape)
    raw = _sc_embed(xf, t_lin, xf.shape[0])
    return jnp.transpose(raw, (2, 0, 1))


# speedup vs baseline: 1.8165x; 1.0216x over previous
import functools
import math

import jax
import jax.numpy as jnp
from jax import lax
from jax.experimental import pallas as pl
from jax.experimental.pallas import tpu as pltpu
from jax.experimental.pallas import tpu_sc as plsc

NC = 2
NS = 16
NW = NC * NS
L = 16
DIM = 32
SCALE = math.sqrt(float(DIM))
CHUNK = 800
XROWS = CHUNK // 50


@functools.partial(jax.jit, static_argnames=("n_total",))
def _sc_embed(x, table, n_total):
    b_per_w = n_total // NW
    n_chunks = b_per_w // CHUNK
    mesh = plsc.VectorSubcoreMesh(
        core_axis_name="c", subcore_axis_name="s",
        num_cores=NC, num_subcores=NS)

    @functools.partial(
        pl.kernel,
        mesh=mesh,
        out_type=jax.ShapeDtypeStruct((50, DIM, n_total // 50), jnp.float32),
        scratch_types=[
            pltpu.VMEM((b_per_w,), jnp.int32),
            pltpu.VMEM((CHUNK, DIM), jnp.float32),
            pltpu.VMEM((CHUNK, DIM), jnp.float32),
            pltpu.VMEM((50, DIM, XROWS), jnp.float32),
            pltpu.VMEM((50, DIM, XROWS), jnp.float32),
            pltpu.SemaphoreType.DMA,
            pltpu.SemaphoreType.DMA,
            pltpu.SemaphoreType.DMA,
            pltpu.SemaphoreType.DMA,
        ],
        compiler_params=pltpu.CompilerParams(use_tc_tiling_on_sc=False,
                                             needs_layout_passes=False),
    )
    def k(x_hbm, table_hbm, out_hbm, idx_v, rows0, rows1, tv0, tv1,
          g0, g1, o0, o1):
        rows = (rows0, rows1)
        tvs = (tv0, tv1)
        gsem = (g0, g1)
        osem = (o0, o1)
        wid = lax.axis_index("s") * NC + lax.axis_index("c")
        base = wid * b_per_w
        lanes = lax.iota(jnp.int32, L)

        pltpu.sync_copy(x_hbm.at[pl.ds(pl.multiple_of(base, 8), b_per_w)],
                        idx_v)

        def idx_slice(c):
            return idx_v.at[pl.ds(c * CHUNK, CHUNK)]

        def start_gather(c, b):
            pltpu.async_copy(table_hbm.at[idx_slice(c)], rows[b], gsem[b])

        def wait_gather(b):
            pltpu.make_async_copy(table_hbm.at[idx_slice(0)], rows[b],
                                  gsem[b]).wait()

        def out_slice(c):
            r0 = wid * n_chunks * XROWS + c * XROWS
            return out_hbm.at[:, :, pl.ds(r0, XROWS)]

        def start_out(c, b):
            pltpu.async_copy(tvs[b], out_slice(c), osem[b])

        def wait_out(c, b):
            pltpu.make_async_copy(tvs[b], out_slice(c), osem[b]).wait()

        def transpose_scale(b):
            rv, tv = rows[b], tvs[b]

            def t_body(t, carry2):
                ridx = lanes * 50 + t

                @plsc.parallel_loop(0, DIM, unroll=8)
                def d_body(d):
                    v = plsc.load_gather(rv, [ridx, jnp.full((L,), d,
                                                             jnp.int32)])
                    tv[t, d, :] = v

                return carry2

            lax.fori_loop(0, 50, t_body, 0)

        start_gather(0, 0)

        def outer(o, carry):
            for b in range(2):
                c = 2 * o + b
                wait_gather(b)

                @pl.when(c + 1 < n_chunks)
                def _():
                    start_gather(c + 1, 1 - b)

                @pl.when(c >= 2)
                def _():
                    wait_out(c - 2, b)

                transpose_scale(b)
                start_out(c, b)
            return carry

        lax.fori_loop(0, n_chunks // 2, outer, 0)
        wait_out(n_chunks - 2, 0)
        wait_out(n_chunks - 1, 1)

    return k(x, table)


_TW = 1920


def _lin_body(in_ref, out_ref):
    pack = 128 // DIM
    for j in range(_TW // 128):
        y = in_ref[:, pl.ds(j * 128, 128)].T
        z = y.reshape(128 // pack, pack, DIM)
        out_ref[pl.ds(j * (128 // pack), 128 // pack), :] = jnp.concatenate(
            [z[:, f, :] for f in range(pack)], axis=-1) * SCALE


def _tc_linearize(tT):
    v = tT.shape[1]
    return pl.pallas_call(
        _lin_body,
        grid=((v + _TW - 1) // _TW,),
        in_specs=[pl.BlockSpec((DIM, _TW), lambda i: (0, i))],
        out_specs=pl.BlockSpec((_TW * DIM // 128, 128), lambda i: (i, 0)),
        out_shape=jax.ShapeDtypeStruct((v * DIM // 128, 128), jnp.float32),
    )(tT)


def kernel(x, table):
    xf = x.reshape(-1)
    t_lin = _tc_linearize(table.T).reshape(table.shape)
    raw = _sc_embed(xf, t_lin, xf.shape[0])
    return jnp.transpose(raw, (2, 0, 1))


# speedup vs baseline: 2.0539x; 1.1307x over previous
import functools
import math

import jax
import jax.numpy as jnp
from jax import lax
from jax.experimental import pallas as pl
from jax.experimental.pallas import tpu as pltpu
from jax.experimental.pallas import tpu_sc as plsc

NC = 2
NS = 16
NW = NC * NS
L = 16
DIM = 32
SCALE = math.sqrt(float(DIM))
CHUNK = 800
XROWS = CHUNK // 50


@functools.partial(jax.jit, static_argnames=("n_total",))
def _sc_embed(x, table, n_total):
    b_per_w = n_total // NW
    n_chunks = b_per_w // CHUNK
    mesh = plsc.VectorSubcoreMesh(
        core_axis_name="c", subcore_axis_name="s",
        num_cores=NC, num_subcores=NS)

    @functools.partial(
        pl.kernel,
        mesh=mesh,
        out_type=jax.ShapeDtypeStruct((50, DIM, n_total // 50), jnp.float32),
        scratch_types=[
            pltpu.VMEM((b_per_w,), jnp.int32),
            pltpu.VMEM((CHUNK, DIM), jnp.float32),
            pltpu.VMEM((CHUNK, DIM), jnp.float32),
            pltpu.VMEM((50, DIM, XROWS), jnp.float32),
            pltpu.VMEM((50, DIM, XROWS), jnp.float32),
            pltpu.SemaphoreType.DMA,
            pltpu.SemaphoreType.DMA,
            pltpu.SemaphoreType.DMA,
            pltpu.SemaphoreType.DMA,
        ],
        compiler_params=pltpu.CompilerParams(use_tc_tiling_on_sc=False,
                                             needs_layout_passes=False),
    )
    def k(x_hbm, table_hbm, out_hbm, idx_v, rows0, rows1, tv0, tv1,
          g0, g1, o0, o1):
        rows = (rows0, rows1)
        tvs = (tv0, tv1)
        gsem = (g0, g1)
        osem = (o0, o1)
        wid = lax.axis_index("s") * NC + lax.axis_index("c")
        base = wid * b_per_w
        lanes = lax.iota(jnp.int32, L)

        pltpu.sync_copy(x_hbm.at[pl.ds(pl.multiple_of(base, 8), b_per_w)],
                        idx_v)

        def idx_slice(c):
            return idx_v.at[pl.ds(c * CHUNK, CHUNK)]

        def start_gather(c, b):
            pltpu.async_copy(table_hbm.at[idx_slice(c)], rows[b], gsem[b])

        def wait_gather(b):
            pltpu.make_async_copy(table_hbm.at[idx_slice(0)], rows[b],
                                  gsem[b]).wait()

        def out_slice(c):
            r0 = wid * n_chunks * XROWS + c * XROWS
            return out_hbm.at[:, :, pl.ds(r0, XROWS)]

        def start_out(c, b):
            pltpu.async_copy(tvs[b], out_slice(c), osem[b])

        def wait_out(c, b):
            pltpu.make_async_copy(tvs[b], out_slice(c), osem[b]).wait()

        def transpose_scale(b):
            rv, tv = rows[b], tvs[b]

            def t_body(t, carry2):
                ridx = lanes * 50 + t

                @plsc.parallel_loop(0, DIM, unroll=8)
                def d_body(d):
                    v = plsc.load_gather(rv, [ridx, jnp.full((L,), d,
                                                             jnp.int32)])
                    tv[t, d, :] = v

                return carry2

            lax.fori_loop(0, 50, t_body, 0)

        start_gather(0, 0)

        def outer(o, carry):
            for b in range(2):
                c = 2 * o + b
                wait_gather(b)

                @pl.when(c + 1 < n_chunks)
                def _():
                    start_gather(c + 1, 1 - b)

                @pl.when(c >= 2)
                def _():
                    wait_out(c - 2, b)

                transpose_scale(b)
                start_out(c, b)
            return carry

        lax.fori_loop(0, n_chunks // 2, outer, 0)
        wait_out(n_chunks - 2, 0)
        wait_out(n_chunks - 1, 1)

    return k(x, table)


_TW = 3840


def _lin_body(in_ref, out_ref):
    pack = 128 // DIM
    for j in range(_TW // 128):
        y = in_ref[:, pl.ds(j * 128, 128)].T
        z = y.reshape(128 // pack, pack, DIM)
        out_ref[pl.ds(j * (128 // pack), 128 // pack), :] = jnp.concatenate(
            [z[:, f, :] for f in range(pack)], axis=-1) * SCALE


def _tc_linearize(tT):
    v = tT.shape[1]
    return pl.pallas_call(
        _lin_body,
        grid=((v + _TW - 1) // _TW,),
        in_specs=[pl.BlockSpec((DIM, _TW), lambda i: (0, i))],
        out_specs=pl.BlockSpec((_TW * DIM // 128, 128), lambda i: (i, 0)),
        out_shape=jax.ShapeDtypeStruct((v * DIM // 128, 128), jnp.float32),
    )(tT)


def kernel(x, table):
    xf = x.reshape(-1)
    t_lin = _tc_linearize(table.T).reshape(table.shape)
    raw = _sc_embed(xf, t_lin, xf.shape[0])
    return jnp.transpose(raw, (2, 0, 1))


# speedup vs baseline: 2.0825x; 1.0139x over previous
import functools
import math

import jax
import jax.numpy as jnp
from jax import lax
from jax.experimental import pallas as pl
from jax.experimental.pallas import tpu as pltpu
from jax.experimental.pallas import tpu_sc as plsc

NC = 2
NS = 16
NW = NC * NS
L = 16
DIM = 32
SCALE = math.sqrt(float(DIM))
CHUNK = 800
XROWS = CHUNK // 50


@functools.partial(jax.jit, static_argnames=("n_total",))
def _sc_embed(x, table, n_total):
    b_per_w = n_total // NW
    n_chunks = b_per_w // CHUNK
    mesh = plsc.VectorSubcoreMesh(
        core_axis_name="c", subcore_axis_name="s",
        num_cores=NC, num_subcores=NS)

    @functools.partial(
        pl.kernel,
        mesh=mesh,
        out_type=jax.ShapeDtypeStruct((50, DIM, n_total // 50), jnp.float32),
        scratch_types=[
            pltpu.VMEM((b_per_w,), jnp.int32),
            pltpu.VMEM((CHUNK, DIM), jnp.float32),
            pltpu.VMEM((CHUNK, DIM), jnp.float32),
            pltpu.VMEM((50, DIM, XROWS), jnp.float32),
            pltpu.VMEM((50, DIM, XROWS), jnp.float32),
            pltpu.SemaphoreType.DMA,
            pltpu.SemaphoreType.DMA,
            pltpu.SemaphoreType.DMA,
            pltpu.SemaphoreType.DMA,
        ],
        compiler_params=pltpu.CompilerParams(use_tc_tiling_on_sc=False,
                                             needs_layout_passes=False),
    )
    def k(x_hbm, table_hbm, out_hbm, idx_v, rows0, rows1, tv0, tv1,
          g0, g1, o0, o1):
        rows = (rows0, rows1)
        tvs = (tv0, tv1)
        gsem = (g0, g1)
        osem = (o0, o1)
        wid = lax.axis_index("s") * NC + lax.axis_index("c")
        base = wid * b_per_w
        lanes = lax.iota(jnp.int32, L)

        pltpu.sync_copy(x_hbm.at[pl.ds(pl.multiple_of(base, 8), b_per_w)],
                        idx_v)

        def idx_slice(c):
            return idx_v.at[pl.ds(c * CHUNK, CHUNK)]

        def start_gather(c, b):
            pltpu.async_copy(table_hbm.at[idx_slice(c)], rows[b], gsem[b])

        def wait_gather(b):
            pltpu.make_async_copy(table_hbm.at[idx_slice(0)], rows[b],
                                  gsem[b]).wait()

        def out_slice(c):
            r0 = wid * n_chunks * XROWS + c * XROWS
            return out_hbm.at[:, :, pl.ds(r0, XROWS)]

        def start_out(c, b):
            pltpu.async_copy(tvs[b], out_slice(c), osem[b])

        def wait_out(c, b):
            pltpu.make_async_copy(tvs[b], out_slice(c), osem[b]).wait()

        def transpose_scale(b):
            rv, tv = rows[b], tvs[b]

            def t_body(t, carry2):
                ridx = lanes * 50 + t

                @plsc.parallel_loop(0, DIM, unroll=8)
                def d_body(d):
                    v = plsc.load_gather(rv, [ridx, jnp.full((L,), d,
                                                             jnp.int32)])
                    tv[t, d, :] = v

                return carry2

            lax.fori_loop(0, 50, t_body, 0)

        start_gather(0, 0)

        def outer(o, carry):
            for b in range(2):
                c = 2 * o + b
                wait_gather(b)

                @pl.when(c + 1 < n_chunks)
                def _():
                    start_gather(c + 1, 1 - b)

                @pl.when(c >= 2)
                def _():
                    wait_out(c - 2, b)

                transpose_scale(b)
                start_out(c, b)
            return carry

        lax.fori_loop(0, n_chunks // 2, outer, 0)
        wait_out(n_chunks - 2, 0)
        wait_out(n_chunks - 1, 1)

    return k(x, table)


_TW = 7680


def _lin_body(in_ref, out_ref):
    pack = 128 // DIM
    for j in range(_TW // 128):
        y = in_ref[:, pl.ds(j * 128, 128)].T
        z = y.reshape(128 // pack, pack, DIM)
        out_ref[pl.ds(j * (128 // pack), 128 // pack), :] = jnp.concatenate(
            [z[:, f, :] for f in range(pack)], axis=-1) * SCALE


def _tc_linearize(tT):
    v = tT.shape[1]
    return pl.pallas_call(
        _lin_body,
        grid=((v + _TW - 1) // _TW,),
        in_specs=[pl.BlockSpec((DIM, _TW), lambda i: (0, i))],
        out_specs=pl.BlockSpec((_TW * DIM // 128, 128), lambda i: (i, 0)),
        out_shape=jax.ShapeDtypeStruct((v * DIM // 128, 128), jnp.float32),
    )(tT)


def kernel(x, table):
    xf = x.reshape(-1)
    t_lin = _tc_linearize(table.T).reshape(table.shape)
    raw = _sc_embed(xf, t_lin, xf.shape[0])
    return jnp.transpose(raw, (2, 0, 1))
